# baseline (speedup 1.0000x reference)
import functools

import jax
import jax.numpy as jnp
from jax import lax
from jax.experimental import pallas as pl
from jax.experimental.pallas import tpu as pltpu
from jax.experimental.pallas import tpu_sc as plsc

CHANNELS = 64
NBINS = 8
TAIL = 3.0
MINW = 0.01
MINH = 0.01
MIND = 0.01
EPS = 1e-6
N = 65536

NC = 2
NS = 16
NW = NC * NS
ROWS_PER_W = N // NW
CHUNK = 256
NCHUNK = ROWS_PER_W // CHUNK
NGROUP = CHANNELS // 16
TSZ = CHANNELS * NBINS


def _prep_tables(widths, heights, derivatives):
    w = jax.nn.softmax(widths, axis=-1)
    w = w * (1 - MINW * NBINS) + MINW
    h = jax.nn.softmax(heights, axis=-1)
    h = h * (1 - MINH * NBINS) + MINH
    d = jax.nn.softplus(derivatives) + MIND
    cumw = jnp.pad(jnp.cumsum(w, axis=-1), ((0, 0), (1, 0)))
    cumh = jnp.pad(jnp.cumsum(h, axis=-1), ((0, 0), (1, 0)))
    cumw = cumw / jnp.clip(cumw[..., -1:], EPS, None)
    cumh = cumh / jnp.clip(cumh[..., -1:], EPS, None)
    cumw = 2 * TAIL * cumw - TAIL
    cumh = 2 * TAIL * cumh - TAIL
    bw = cumw[:, 1:] - cumw[:, :-1]
    bh = cumh[:, 1:] - cumh[:, :-1]
    iw = cumw[:, :NBINS]
    ih = cumh[:, :NBINS]
    delta = bh / bw
    inv_bw = 1.0 / bw
    dl = d[:, :NBINS]
    s = dl + d[:, 1:] - 2 * delta
    tabs = jnp.concatenate(
        [t.T.reshape(-1) for t in (iw, inv_bw, ih, bh, delta, dl, s)]
    )
    knots_t = cumw[:, 1:NBINS].T.reshape(-1)
    return knots_t, tabs


def _sc_spline(x, knots_t, tabs):
    mesh = plsc.VectorSubcoreMesh(core_axis_name="c", subcore_axis_name="s")

    @functools.partial(
        pl.kernel,
        out_type=jax.ShapeDtypeStruct((N, CHANNELS), jnp.float32),
        mesh=mesh,
        compiler_params=pltpu.CompilerParams(
            needs_layout_passes=False, use_tc_tiling_on_sc=False
        ),
        scratch_types=[
            pltpu.VMEM((CHUNK, CHANNELS), jnp.float32),
            pltpu.VMEM((CHUNK, CHANNELS), jnp.float32),
            pltpu.VMEM((7 * CHANNELS,), jnp.float32),
            pltpu.VMEM((7 * TSZ,), jnp.float32),
            pltpu.SemaphoreType.DMA,
            pltpu.SemaphoreType.DMA,
            pltpu.SemaphoreType.DMA,
            pltpu.SemaphoreType.DMA,
        ],
    )
    def k(x_hbm, kn_hbm, tab_hbm, out_hbm, buf0, buf1,
          kn_v, tab_v, in_sem0, in_sem1, out_sem0, out_sem1):
        wid = lax.axis_index("s") * NC + lax.axis_index("c")
        row0 = wid * ROWS_PER_W
        pltpu.sync_copy(kn_hbm, kn_v)
        pltpu.sync_copy(tab_hbm, tab_v)

        bufs = [buf0, buf1]
        in_sems = [in_sem0, in_sem1]
        out_sems = [out_sem0, out_sem1]
        lane = lax.iota(jnp.int32, 16)
        knot_vs = [
            [kn_v[pl.ds(kk * CHANNELS + 16 * j, 16)] for kk in range(7)]
            for j in range(NGROUP)
        ]
        chan = [lane + 16 * j for j in range(NGROUP)]
        step = jnp.full((16,), CHANNELS, jnp.int32)
        zero = jnp.full((16,), 0, jnp.int32)

        def process(buf):
            def one_row(r):
                for j in range(NGROUP):
                    xv = buf[r, pl.ds(16 * j, 16)]
                    xc = jnp.clip(xv, -TAIL + EPS, TAIL - EPS)
                    idx = chan[j]
                    for kk in range(7):
                        idx = idx + jnp.where(
                            xc >= knot_vs[j][kk], step, zero
                        )
                    gat = lambda p: plsc.load_gather(tab_v, [idx + p * TSZ])
                    iw = gat(0)
                    inv_bw = gat(1)
                    ih = gat(2)
                    bh = gat(3)
                    delta = gat(4)
                    dl = gat(5)
                    s = gat(6)
                    theta = jnp.clip((xc - iw) * inv_bw, 0.0, 1.0)
                    omt = 1.0 - theta
                    num = bh * (theta * (delta * theta + dl * omt))
                    den = delta + s * (theta * omt)
                    y = ih + num / den
                    out = jnp.where((xv > TAIL) | (xv < -TAIL), xv, y)
                    buf[r, pl.ds(16 * j, 16)] = out

            lax.fori_loop(
                0, CHUNK, lambda r, carry: (one_row(r), 0)[1], 0, unroll=4
            )

        in_cps = [None] * NCHUNK
        out_cps = [None] * NCHUNK

        def start_in(c):
            b = c % 2
            in_cps[c] = pltpu.async_copy(
                x_hbm.at[pl.ds(row0 + c * CHUNK, CHUNK)], bufs[b],
                in_sems[b],
            )

        def start_out(c):
            b = c % 2
            out_cps[c] = pltpu.async_copy(
                bufs[b], out_hbm.at[pl.ds(row0 + c * CHUNK, CHUNK)],
                out_sems[b],
            )

        start_in(0)
        for c in range(NCHUNK):
            in_cps[c].wait()
            if c + 1 < NCHUNK:
                if c >= 1:
                    out_cps[c - 1].wait()
                start_in(c + 1)
            process(bufs[c % 2])
            start_out(c)
        out_cps[NCHUNK - 2].wait()
        out_cps[NCHUNK - 1].wait()

    return k(x, knots_t, tabs)


@jax.jit
def kernel(inputs, widths, heights, derivatives):
    knots_t, tabs = _prep_tables(widths, heights, derivatives)
    return _sc_spline(inputs, knots_t, tabs)


# speedup vs baseline: 1.3215x; 1.3215x over previous
import functools

import jax
import jax.numpy as jnp
from jax import lax
from jax.experimental import pallas as pl
from jax.experimental.pallas import tpu as pltpu
from jax.experimental.pallas import tpu_sc as plsc

CHANNELS = 64
NBINS = 8
TAIL = 3.0
MINW = 0.01
MINH = 0.01
MIND = 0.01
EPS = 1e-6
N = 65536

NC = 2
NS = 16
NW = NC * NS
ROWS_PER_W = N // NW
CHUNK = 256
NCHUNK = ROWS_PER_W // CHUNK
NGROUP = CHANNELS // 16
TSZ = CHANNELS * NBINS


def _prep_tables(widths, heights, derivatives):
    w = jax.nn.softmax(widths, axis=-1)
    w = w * (1 - MINW * NBINS) + MINW
    h = jax.nn.softmax(heights, axis=-1)
    h = h * (1 - MINH * NBINS) + MINH
    d = jax.nn.softplus(derivatives) + MIND
    cumw = jnp.pad(jnp.cumsum(w, axis=-1), ((0, 0), (1, 0)))
    cumh = jnp.pad(jnp.cumsum(h, axis=-1), ((0, 0), (1, 0)))
    cumw = cumw / jnp.clip(cumw[..., -1:], EPS, None)
    cumh = cumh / jnp.clip(cumh[..., -1:], EPS, None)
    cumw = 2 * TAIL * cumw - TAIL
    cumh = 2 * TAIL * cumh - TAIL
    bw = cumw[:, 1:] - cumw[:, :-1]
    bh = cumh[:, 1:] - cumh[:, :-1]
    iw = cumw[:, :NBINS]
    ih = cumh[:, :NBINS]
    delta = bh / bw
    inv_bw = 1.0 / bw
    dl = d[:, :NBINS]
    s = dl + d[:, 1:] - 2 * delta
    tabs = jnp.concatenate(
        [t.T.reshape(-1) for t in (iw, inv_bw, ih, bh, delta, dl, s)]
    )
    knots_t = cumw[:, 1:NBINS].T.reshape(-1)
    return knots_t, tabs


def _sc_spline(x, knots_t, tabs):
    mesh = plsc.VectorSubcoreMesh(core_axis_name="c", subcore_axis_name="s")

    @functools.partial(
        pl.kernel,
        out_type=jax.ShapeDtypeStruct((N, CHANNELS), jnp.float32),
        mesh=mesh,
        compiler_params=pltpu.CompilerParams(
            needs_layout_passes=False, use_tc_tiling_on_sc=False
        ),
        scratch_types=[
            pltpu.VMEM((CHUNK, CHANNELS), jnp.float32),
            pltpu.VMEM((CHUNK, CHANNELS), jnp.float32),
            pltpu.VMEM((7 * CHANNELS,), jnp.float32),
            pltpu.VMEM((7 * TSZ,), jnp.float32),
            pltpu.SemaphoreType.DMA,
            pltpu.SemaphoreType.DMA,
            pltpu.SemaphoreType.DMA,
            pltpu.SemaphoreType.DMA,
        ],
    )
    def k(x_hbm, kn_hbm, tab_hbm, out_hbm, buf0, buf1,
          kn_v, tab_v, in_sem0, in_sem1, out_sem0, out_sem1):
        wid = lax.axis_index("s") * NC + lax.axis_index("c")
        row0 = wid * ROWS_PER_W
        pltpu.sync_copy(kn_hbm, kn_v)
        pltpu.sync_copy(tab_hbm, tab_v)

        bufs = [buf0, buf1]
        in_sems = [in_sem0, in_sem1]
        out_sems = [out_sem0, out_sem1]
        lane = lax.iota(jnp.int32, 16)
        knot_vs = [
            [kn_v[pl.ds(kk * CHANNELS + 16 * j, 16)] for kk in range(7)]
            for j in range(NGROUP)
        ]
        chan = [lane + 16 * j for j in range(NGROUP)]
        step = jnp.full((16,), CHANNELS, jnp.int32)
        zero = jnp.full((16,), 0, jnp.int32)

        def process(buf):
            def one_row(r):
                for j in range(NGROUP):
                    xv = buf[r, pl.ds(16 * j, 16)]
                    xc = jnp.clip(xv, -TAIL + EPS, TAIL - EPS)
                    idx = chan[j]
                    for kk in range(7):
                        idx = idx + jnp.where(
                            xc >= knot_vs[j][kk], step, zero
                        )
                    gat = lambda p: plsc.load_gather(tab_v, [idx + p * TSZ])
                    iw = gat(0)
                    inv_bw = gat(1)
                    ih = gat(2)
                    bh = gat(3)
                    delta = gat(4)
                    dl = gat(5)
                    s = gat(6)
                    theta = jnp.clip((xc - iw) * inv_bw, 0.0, 1.0)
                    omt = 1.0 - theta
                    num = bh * (theta * (delta * theta + dl * omt))
                    den = delta + s * (theta * omt)
                    y = ih + num / den
                    out = jnp.where((xv > TAIL) | (xv < -TAIL), xv, y)
                    buf[r, pl.ds(16 * j, 16)] = out

            lax.fori_loop(
                0, CHUNK, lambda r, carry: (one_row(r), 0)[1], 0
            )

        in_cps = [None] * NCHUNK
        out_cps = [None] * NCHUNK

        def start_in(c):
            b = c % 2
            in_cps[c] = pltpu.async_copy(
                x_hbm.at[pl.ds(row0 + c * CHUNK, CHUNK)], bufs[b],
                in_sems[b],
            )

        def start_out(c):
            b = c % 2
            out_cps[c] = pltpu.async_copy(
                bufs[b], out_hbm.at[pl.ds(row0 + c * CHUNK, CHUNK)],
                out_sems[b],
            )

        start_in(0)
        for c in range(NCHUNK):
            in_cps[c].wait()
            if c + 1 < NCHUNK:
                if c >= 1:
                    out_cps[c - 1].wait()
                start_in(c + 1)
            process(bufs[c % 2])
            start_out(c)
        out_cps[NCHUNK - 2].wait()
        out_cps[NCHUNK - 1].wait()

    return k(x, knots_t, tabs)


@jax.jit
def kernel(inputs, widths, heights, derivatives):
    knots_t, tabs = _prep_tables(widths, heights, derivatives)
    return _sc_spline(inputs, knots_t, tabs)


# speedup vs baseline: 1.4568x; 1.1023x over previous
import functools

import jax
import jax.numpy as jnp
from jax import lax
from jax.experimental import pallas as pl
from jax.experimental.pallas import tpu as pltpu
from jax.experimental.pallas import tpu_sc as plsc

CHANNELS = 64
NBINS = 8
TAIL = 3.0
MINW = 0.01
MINH = 0.01
MIND = 0.01
EPS = 1e-6
N = 65536

NC = 2
NS = 16
NW = NC * NS
ROWS_PER_W = N // NW
CHUNK = 256
NCHUNK = ROWS_PER_W // CHUNK
NGROUP = CHANNELS // 16
TSZ = CHANNELS * NBINS


def _prep_tables(widths, heights, derivatives):
    w = jax.nn.softmax(widths, axis=-1)
    w = w * (1 - MINW * NBINS) + MINW
    h = jax.nn.softmax(heights, axis=-1)
    h = h * (1 - MINH * NBINS) + MINH
    d = jax.nn.softplus(derivatives) + MIND
    cumw = jnp.pad(jnp.cumsum(w, axis=-1), ((0, 0), (1, 0)))
    cumh = jnp.pad(jnp.cumsum(h, axis=-1), ((0, 0), (1, 0)))
    cumw = cumw / jnp.clip(cumw[..., -1:], EPS, None)
    cumh = cumh / jnp.clip(cumh[..., -1:], EPS, None)
    cumw = 2 * TAIL * cumw - TAIL
    cumh = 2 * TAIL * cumh - TAIL
    bw = cumw[:, 1:] - cumw[:, :-1]
    bh = cumh[:, 1:] - cumh[:, :-1]
    iw = cumw[:, :NBINS]
    ih = cumh[:, :NBINS]
    delta = bh / bw
    inv_bw = 1.0 / bw
    dl = d[:, :NBINS]
    s = dl + d[:, 1:] - 2 * delta
    tabs = jnp.concatenate(
        [t.T.reshape(-1) for t in (iw, inv_bw, ih, bh, delta, dl, s)]
    )
    knots_t = cumw[:, 1:NBINS].T.reshape(-1)
    return knots_t, tabs


def _sc_spline(x, knots_t, tabs):
    mesh = plsc.VectorSubcoreMesh(core_axis_name="c", subcore_axis_name="s")

    @functools.partial(
        pl.kernel,
        out_type=jax.ShapeDtypeStruct((N, CHANNELS), jnp.float32),
        mesh=mesh,
        compiler_params=pltpu.CompilerParams(
            needs_layout_passes=False, use_tc_tiling_on_sc=True
        ),
        scratch_types=[
            pltpu.VMEM((CHUNK, CHANNELS), jnp.float32),
            pltpu.VMEM((CHUNK, CHANNELS), jnp.float32),
            pltpu.VMEM((7 * CHANNELS,), jnp.float32),
            pltpu.VMEM((7 * TSZ,), jnp.float32),
            pltpu.SemaphoreType.DMA,
            pltpu.SemaphoreType.DMA,
            pltpu.SemaphoreType.DMA,
            pltpu.SemaphoreType.DMA,
        ],
    )
    def k(x_hbm, kn_hbm, tab_hbm, out_hbm, buf0, buf1,
          kn_v, tab_v, in_sem0, in_sem1, out_sem0, out_sem1):
        wid = lax.axis_index("s") * NC + lax.axis_index("c")
        row0 = wid * ROWS_PER_W
        pltpu.sync_copy(kn_hbm, kn_v)
        pltpu.sync_copy(tab_hbm, tab_v)

        bufs = [buf0, buf1]
        in_sems = [in_sem0, in_sem1]
        out_sems = [out_sem0, out_sem1]
        lane = lax.iota(jnp.int32, 16)
        knot_vs = [
            [kn_v[pl.ds(kk * CHANNELS + 16 * j, 16)] for kk in range(7)]
            for j in range(NGROUP)
        ]
        chan = [lane + 16 * j for j in range(NGROUP)]
        step = jnp.full((16,), CHANNELS, jnp.int32)
        zero = jnp.full((16,), 0, jnp.int32)

        def process(buf):
            def one_row(r):
                for j in range(NGROUP):
                    xv = buf[r, pl.ds(16 * j, 16)]
                    xc = jnp.clip(xv, -TAIL + EPS, TAIL - EPS)
                    idx = chan[j]
                    for kk in range(7):
                        idx = idx + jnp.where(
                            xc >= knot_vs[j][kk], step, zero
                        )
                    gat = lambda p: plsc.load_gather(tab_v, [idx + p * TSZ])
                    iw = gat(0)
                    inv_bw = gat(1)
                    ih = gat(2)
                    bh = gat(3)
                    delta = gat(4)
                    dl = gat(5)
                    s = gat(6)
                    theta = jnp.clip((xc - iw) * inv_bw, 0.0, 1.0)
                    omt = 1.0 - theta
                    num = bh * (theta * (delta * theta + dl * omt))
                    den = delta + s * (theta * omt)
                    y = ih + num / den
                    out = jnp.where((xv > TAIL) | (xv < -TAIL), xv, y)
                    buf[r, pl.ds(16 * j, 16)] = out

            lax.fori_loop(
                0, CHUNK, lambda r, carry: (one_row(r), 0)[1], 0
            )

        in_cps = [None] * NCHUNK
        out_cps = [None] * NCHUNK

        def start_in(c):
            b = c % 2
            in_cps[c] = pltpu.async_copy(
                x_hbm.at[pl.ds(row0 + c * CHUNK, CHUNK)], bufs[b],
                in_sems[b],
            )

        def start_out(c):
            b = c % 2
            out_cps[c] = pltpu.async_copy(
                bufs[b], out_hbm.at[pl.ds(row0 + c * CHUNK, CHUNK)],
                out_sems[b],
            )

        start_in(0)
        for c in range(NCHUNK):
            in_cps[c].wait()
            if c + 1 < NCHUNK:
                if c >= 1:
                    out_cps[c - 1].wait()
                start_in(c + 1)
            process(bufs[c % 2])
            start_out(c)
        out_cps[NCHUNK - 2].wait()
        out_cps[NCHUNK - 1].wait()

    return k(x, knots_t, tabs)


@jax.jit
def kernel(inputs, widths, heights, derivatives):
    knots_t, tabs = _prep_tables(widths, heights, derivatives)
    return _sc_spline(inputs, knots_t, tabs)


# speedup vs baseline: 1.4954x; 1.0265x over previous
import functools

import jax
import jax.numpy as jnp
from jax import lax
from jax.experimental import pallas as pl
from jax.experimental.pallas import tpu as pltpu
from jax.experimental.pallas import tpu_sc as plsc

CHANNELS = 64
NBINS = 8
TAIL = 3.0
MINW = 0.01
MINH = 0.01
MIND = 0.01
EPS = 1e-6
N = 65536

NC = 2
NS = 16
NW = NC * NS
ROWS_PER_W = N // NW
CHUNK = 256
NCHUNK = ROWS_PER_W // CHUNK
NGROUP = CHANNELS // 16
TSZ = CHANNELS * NBINS


def _prep_tables(widths, heights, derivatives):
    w = jax.nn.softmax(widths, axis=-1)
    w = w * (1 - MINW * NBINS) + MINW
    h = jax.nn.softmax(heights, axis=-1)
    h = h * (1 - MINH * NBINS) + MINH
    d = jax.nn.softplus(derivatives) + MIND
    cumw = jnp.pad(jnp.cumsum(w, axis=-1), ((0, 0), (1, 0)))
    cumh = jnp.pad(jnp.cumsum(h, axis=-1), ((0, 0), (1, 0)))
    cumw = cumw / jnp.clip(cumw[..., -1:], EPS, None)
    cumh = cumh / jnp.clip(cumh[..., -1:], EPS, None)
    cumw = 2 * TAIL * cumw - TAIL
    cumh = 2 * TAIL * cumh - TAIL
    bw = cumw[:, 1:] - cumw[:, :-1]
    bh = cumh[:, 1:] - cumh[:, :-1]
    iw = cumw[:, :NBINS]
    ih = cumh[:, :NBINS]
    delta = bh / bw
    inv_bw = 1.0 / bw
    dl = d[:, :NBINS]
    s = dl + d[:, 1:] - 2 * delta
    c0 = -iw * inv_bw
    tabs = jnp.concatenate(
        [t.T.reshape(-1) for t in (inv_bw, c0, ih, bh, dl, s)]
    )
    knots_t = cumw[:, 1:NBINS].T.reshape(-1)
    return knots_t, tabs


def _sc_spline(x, knots_t, tabs):
    mesh = plsc.VectorSubcoreMesh(core_axis_name="c", subcore_axis_name="s")

    @functools.partial(
        pl.kernel,
        out_type=jax.ShapeDtypeStruct((N, CHANNELS), jnp.float32),
        mesh=mesh,
        compiler_params=pltpu.CompilerParams(
            needs_layout_passes=False, use_tc_tiling_on_sc=True
        ),
        scratch_types=[
            pltpu.VMEM((CHUNK, CHANNELS), jnp.float32),
            pltpu.VMEM((CHUNK, CHANNELS), jnp.float32),
            pltpu.VMEM((7 * CHANNELS,), jnp.float32),
            pltpu.VMEM((TSZ,), jnp.float32),
            pltpu.VMEM((TSZ,), jnp.float32),
            pltpu.VMEM((TSZ,), jnp.float32),
            pltpu.VMEM((TSZ,), jnp.float32),
            pltpu.VMEM((TSZ,), jnp.float32),
            pltpu.VMEM((TSZ,), jnp.float32),
            pltpu.SemaphoreType.DMA,
            pltpu.SemaphoreType.DMA,
            pltpu.SemaphoreType.DMA,
            pltpu.SemaphoreType.DMA,
        ],
    )
    def k(x_hbm, kn_hbm, tab_hbm, out_hbm, buf0, buf1,
          kn_v, t_a, t_c0, t_ih, t_bh, t_dl, t_s,
          in_sem0, in_sem1, out_sem0, out_sem1):
        wid = lax.axis_index("s") * NC + lax.axis_index("c")
        row0 = wid * ROWS_PER_W
        pltpu.sync_copy(kn_hbm, kn_v)
        tab_refs = [t_a, t_c0, t_ih, t_bh, t_dl, t_s]
        for p, ref in enumerate(tab_refs):
            pltpu.sync_copy(tab_hbm.at[pl.ds(p * TSZ, TSZ)], ref)

        bufs = [buf0, buf1]
        in_sems = [in_sem0, in_sem1]
        out_sems = [out_sem0, out_sem1]
        lane = lax.iota(jnp.int32, 16)
        knot_vs = [
            [kn_v[pl.ds(kk * CHANNELS + 16 * j, 16)] for kk in range(7)]
            for j in range(NGROUP)
        ]
        chan = [lane + 16 * j for j in range(NGROUP)]
        step = jnp.full((16,), CHANNELS, jnp.int32)
        zero = jnp.full((16,), 0, jnp.int32)

        def process(buf):
            def one_row(r):
                for j in range(NGROUP):
                    xv = buf[r, pl.ds(16 * j, 16)]
                    hits = [
                        jnp.where(xv >= knot_vs[j][kk], step, zero)
                        for kk in range(7)
                    ]
                    h01 = hits[0] + hits[1]
                    h23 = hits[2] + hits[3]
                    h45 = hits[4] + hits[5]
                    idx = (h01 + h23) + (h45 + hits[6]) + chan[j]
                    a = plsc.load_gather(t_a, [idx])
                    c0 = plsc.load_gather(t_c0, [idx])
                    ih = plsc.load_gather(t_ih, [idx])
                    bh = plsc.load_gather(t_bh, [idx])
                    dl = plsc.load_gather(t_dl, [idx])
                    s = plsc.load_gather(t_s, [idx])
                    theta = xv * a + c0
                    omt = 1.0 - theta
                    delta = bh * a
                    num = bh * (theta * (delta * theta + dl * omt))
                    den = delta + s * (theta * omt)
                    y = ih + num / den
                    out = jnp.where(jnp.abs(xv) > TAIL, xv, y)
                    buf[r, pl.ds(16 * j, 16)] = out

            lax.fori_loop(
                0, CHUNK, lambda r, carry: (one_row(r), 0)[1], 0
            )

        in_cps = [None] * NCHUNK
        out_cps = [None] * NCHUNK

        def start_in(c):
            b = c % 2
            in_cps[c] = pltpu.async_copy(
                x_hbm.at[pl.ds(row0 + c * CHUNK, CHUNK)], bufs[b],
                in_sems[b],
            )

        def start_out(c):
            b = c % 2
            out_cps[c] = pltpu.async_copy(
                bufs[b], out_hbm.at[pl.ds(row0 + c * CHUNK, CHUNK)],
                out_sems[b],
            )

        start_in(0)
        for c in range(NCHUNK):
            in_cps[c].wait()
            if c + 1 < NCHUNK:
                if c >= 1:
                    out_cps[c - 1].wait()
                start_in(c + 1)
            process(bufs[c % 2])
            start_out(c)
        out_cps[NCHUNK - 2].wait()
        out_cps[NCHUNK - 1].wait()

    return k(x, knots_t, tabs)


@jax.jit
def kernel(inputs, widths, heights, derivatives):
    knots_t, tabs = _prep_tables(widths, heights, derivatives)
    return _sc_spline(inputs, knots_t, tabs)
